# Initial kernel scaffold; baseline (speedup 1.0000x reference)
import jax
import jax.numpy as jnp
from jax.experimental import pallas as pl


def kernel(input, value_proj_w, value_proj_b, offset_mask_w, offset_mask_b, output_proj_w, output_proj_b):
    raise NotImplementedError("write your pallas kernel here")


# speedup vs baseline: 10.9121x; 10.9121x over previous
import functools

import jax
import jax.numpy as jnp
import numpy as np
from jax import lax
from jax.experimental import pallas as pl
from jax.experimental.pallas import tpu as pltpu
from jax.experimental.pallas import tpu_sc as plsc

C = 128
G = 4
GC = 32
KS = 3
PTS = 27
D, H, W = 8, 24, 24
L = D * H * W
NCORNER = 8
LANES = 128
NC, NS = 2, 16
NW = NC * NS
PER_W = L // NW


def _om_permutation():
    perm = np.zeros((4 * LANES,), dtype=np.int32)
    live = np.zeros((4 * LANES,), dtype=bool)
    for t in range(4):
        for j in range(G * PTS):
            g, p = j // PTS, j % PTS
            perm[t * LANES + j] = g * 108 + (p * 3 + t if t < 3 else 81 + p)
            live[t * LANES + j] = True
    return perm, live


_PERM, _LIVE = _om_permutation()


def _prep_body(x_ref, vwT_ref, vb_ref, omT_ref, omb_ref, value_ref, idx_ref, w_ref):
    blk = pl.program_id(0)
    bl = x_ref.shape[0]
    x = x_ref[...]
    value_ref[...] = (
        jnp.dot(x, vwT_ref[...], preferred_element_type=jnp.float32) + vb_ref[...]
    )
    om = jnp.dot(x, omT_ref[...], preferred_element_type=jnp.float32) + omb_ref[...]
    od = om[:, 0 * LANES:1 * LANES]
    oh = om[:, 1 * LANES:2 * LANES]
    ow = om[:, 2 * LANES:3 * LANES]
    mk = om[:, 3 * LANES:4 * LANES]

    lane = lax.broadcasted_iota(jnp.int32, (bl, LANES), 1)
    g_l = jnp.minimum(lane // PTS, G - 1)
    p_l = lane % PTS
    kd = (p_l // 9).astype(jnp.float32)
    kh = ((p_l // 3) % 3).astype(jnp.float32)
    kw = (p_l % 3).astype(jnp.float32)

    lglob = blk * bl + lax.broadcasted_iota(jnp.int32, (bl, LANES), 0)
    base_d = (lglob // (H * W)).astype(jnp.float32)
    base_h = ((lglob // W) % H).astype(jnp.float32)
    base_w = (lglob % W).astype(jnp.float32)

    loc_d = base_d - 1.0 + kd + od
    loc_h = base_h - 1.0 + kh + oh
    loc_w = base_w - 1.0 + kw + ow
    d0f = jnp.floor(loc_d)
    h0f = jnp.floor(loc_h)
    w0f = jnp.floor(loc_w)
    fd = loc_d - d0f
    fh = loc_h - h0f
    fw = loc_w - w0f
    d0 = d0f.astype(jnp.int32)
    h0 = h0f.astype(jnp.int32)
    w0 = w0f.astype(jnp.int32)

    corner = 0
    for a in (0, 1):
        wd = fd if a else (1.0 - fd)
        di = d0 + a
        vd = (di >= 0) & (di < D)
        cd = jnp.clip(di, 0, D - 1)
        for b in (0, 1):
            wh = fh if b else (1.0 - fh)
            hi = h0 + b
            vh = (hi >= 0) & (hi < H)
            ch = jnp.clip(hi, 0, H - 1)
            for c in (0, 1):
                ww = fw if c else (1.0 - fw)
                wi = w0 + c
                valid = vd & vh & (wi >= 0) & (wi < W)
                cw = jnp.clip(wi, 0, W - 1)
                wt = wd * wh * ww * mk * valid.astype(jnp.float32)
                ci = (cd * (H * W) + ch * W + cw) * G + g_l
                idx_ref[:, corner, :] = ci
                w_ref[:, corner, :] = wt
                corner += 1


def _prep(x, vwT, vb, omT_pad, omb_pad):
    bl = 512
    grid = L // bl
    return pl.pallas_call(
        _prep_body,
        grid=(grid,),
        in_specs=[
            pl.BlockSpec((bl, C), lambda i: (i, 0)),
            pl.BlockSpec((C, C), lambda i: (0, 0)),
            pl.BlockSpec((C,), lambda i: (0,)),
            pl.BlockSpec((C, 4 * LANES), lambda i: (0, 0)),
            pl.BlockSpec((4 * LANES,), lambda i: (0,)),
        ],
        out_specs=[
            pl.BlockSpec((bl, C), lambda i: (i, 0)),
            pl.BlockSpec((bl, NCORNER, LANES), lambda i: (i, 0, 0)),
            pl.BlockSpec((bl, NCORNER, LANES), lambda i: (i, 0, 0)),
        ],
        out_shape=[
            jax.ShapeDtypeStruct((L, C), jnp.float32),
            jax.ShapeDtypeStruct((L, NCORNER, LANES), jnp.int32),
            jax.ShapeDtypeStruct((L, NCORNER, LANES), jnp.float32),
        ],
    )(x, vwT, vb, omT_pad, omb_pad)


def _sc_body(value_hbm, idx_hbm, w_hbm, out_hbm, idx_v, w_v, rows_v, out_v, gsem):
    wid = lax.axis_index("s") * NC + lax.axis_index("c")
    base = wid * PER_W

    def per_loc(i, _):
        l = base + i
        pltpu.sync_copy(idx_hbm.at[l], idx_v)
        pltpu.sync_copy(w_hbm.at[l], w_v)
        descs = [
            pltpu.async_copy(value_hbm.at[idx_v.at[cc]], rows_v.at[cc], gsem)
            for cc in range(NCORNER)
        ]
        for dsc in descs:
            dsc.wait()

        def per_corner(cc, accs):
            new = list(accs)
            for chunk in range((G * PTS + 15) // 16):
                w16 = w_v[cc, pl.ds(chunk * 16, 16)]
                for j in range(16):
                    s = chunk * 16 + j
                    if s >= G * PTS:
                        break
                    g = s // PTS
                    wt = w16[j]
                    new[2 * g] = new[2 * g] + wt * rows_v[cc, s, pl.ds(0, 16)]
                    new[2 * g + 1] = (
                        new[2 * g + 1] + wt * rows_v[cc, s, pl.ds(16, 16)]
                    )
            return tuple(new)

        zero = jnp.zeros((16,), jnp.float32)
        accs = lax.fori_loop(0, NCORNER, per_corner, (zero,) * (2 * G))
        for g in range(G):
            out_v[pl.ds(g * GC, 16)] = accs[2 * g]
            out_v[pl.ds(g * GC + 16, 16)] = accs[2 * g + 1]
        pltpu.sync_copy(out_v, out_hbm.at[l])
        return 0

    lax.fori_loop(0, PER_W, per_loc, 0)


def _sc_sample(value_flat, idx, w):
    mesh = plsc.VectorSubcoreMesh(
        core_axis_name="c", subcore_axis_name="s", num_cores=NC, num_subcores=NS
    )
    return pl.kernel(
        _sc_body,
        out_type=jax.ShapeDtypeStruct((L, C), jnp.float32),
        mesh=mesh,
        compiler_params=pltpu.CompilerParams(use_tc_tiling_on_sc=False),
        scratch_types=[
            pltpu.VMEM((NCORNER, LANES), jnp.int32),
            pltpu.VMEM((NCORNER, LANES), jnp.float32),
            pltpu.VMEM((NCORNER, LANES, GC), jnp.float32),
            pltpu.VMEM((C,), jnp.float32),
            pltpu.SemaphoreType.DMA,
        ],
    )(value_flat, idx, w)


def _oproj_body(x_ref, owT_ref, ob_ref, out_ref):
    out_ref[...] = (
        jnp.dot(x_ref[...], owT_ref[...], preferred_element_type=jnp.float32)
        + ob_ref[...]
    )


def _oproj(x, owT, ob):
    bl = 512
    return pl.pallas_call(
        _oproj_body,
        grid=(L // bl,),
        in_specs=[
            pl.BlockSpec((bl, C), lambda i: (i, 0)),
            pl.BlockSpec((C, C), lambda i: (0, 0)),
            pl.BlockSpec((C,), lambda i: (0,)),
        ],
        out_specs=pl.BlockSpec((bl, C), lambda i: (i, 0)),
        out_shape=jax.ShapeDtypeStruct((L, C), jnp.float32),
    )(x, owT, ob)


def kernel(input, value_proj_w, value_proj_b, offset_mask_w, offset_mask_b,
           output_proj_w, output_proj_b):
    n, d, h, w_, c = input.shape
    x = input.reshape(L, C)
    perm = jnp.asarray(_PERM)
    live = jnp.asarray(_LIVE, dtype=jnp.float32)
    omw_pad = offset_mask_w[perm] * live[:, None]
    omb_pad = offset_mask_b[perm] * live

    value, idx, wgt = _prep(
        x, value_proj_w.T, value_proj_b, omw_pad.T, omb_pad
    )
    value_flat = value.reshape(L * G, GC)
    sampled = _sc_sample(value_flat, idx, wgt)
    out = _oproj(sampled, output_proj_w.T, output_proj_b)
    return out.reshape(n, d, h, w_, c)


# speedup vs baseline: 19.4422x; 1.7817x over previous
import functools

import jax
import jax.numpy as jnp
import numpy as np
from jax import lax
from jax.experimental import pallas as pl
from jax.experimental.pallas import tpu as pltpu
from jax.experimental.pallas import tpu_sc as plsc

C = 128
G = 4
GC = 32
KS = 3
PTS = 27
D, H, W = 8, 24, 24
L = D * H * W
NCORNER = 8
LANES = 128
NC, NS = 2, 16
NW = NC * NS
PER_W = L // NW


def _om_permutation():
    perm = np.zeros((4 * LANES,), dtype=np.int32)
    live = np.zeros((4 * LANES,), dtype=bool)
    for t in range(4):
        for j in range(G * PTS):
            g, p = j // PTS, j % PTS
            perm[t * LANES + j] = g * 108 + (p * 3 + t if t < 3 else 81 + p)
            live[t * LANES + j] = True
    return perm, live


_PERM, _LIVE = _om_permutation()


def _prep_body(x_ref, vwT_ref, vb_ref, omT_ref, omb_ref, value_ref, iw_ref):
    blk = pl.program_id(0)
    bl = x_ref.shape[0]
    x = x_ref[...]
    value_ref[...] = (
        jnp.dot(x, vwT_ref[...], preferred_element_type=jnp.float32) + vb_ref[...]
    )
    om = jnp.dot(x, omT_ref[...], preferred_element_type=jnp.float32) + omb_ref[...]
    od = om[:, 0 * LANES:1 * LANES]
    oh = om[:, 1 * LANES:2 * LANES]
    ow = om[:, 2 * LANES:3 * LANES]
    mk = om[:, 3 * LANES:4 * LANES]

    lane = lax.broadcasted_iota(jnp.int32, (bl, LANES), 1)
    g_l = jnp.minimum(lane // PTS, G - 1)
    p_l = lane % PTS
    kd = (p_l // 9).astype(jnp.float32)
    kh = ((p_l // 3) % 3).astype(jnp.float32)
    kw = (p_l % 3).astype(jnp.float32)

    lglob = blk * bl + lax.broadcasted_iota(jnp.int32, (bl, LANES), 0)
    base_d = (lglob // (H * W)).astype(jnp.float32)
    base_h = ((lglob // W) % H).astype(jnp.float32)
    base_w = (lglob % W).astype(jnp.float32)

    loc_d = base_d - 1.0 + kd + od
    loc_h = base_h - 1.0 + kh + oh
    loc_w = base_w - 1.0 + kw + ow
    d0f = jnp.floor(loc_d)
    h0f = jnp.floor(loc_h)
    w0f = jnp.floor(loc_w)
    fd = loc_d - d0f
    fh = loc_h - h0f
    fw = loc_w - w0f
    d0 = d0f.astype(jnp.int32)
    h0 = h0f.astype(jnp.int32)
    w0 = w0f.astype(jnp.int32)

    corner = 0
    for a in (0, 1):
        wd = fd if a else (1.0 - fd)
        di = d0 + a
        vd = (di >= 0) & (di < D)
        cd = jnp.clip(di, 0, D - 1)
        for b in (0, 1):
            wh = fh if b else (1.0 - fh)
            hi = h0 + b
            vh = (hi >= 0) & (hi < H)
            ch = jnp.clip(hi, 0, H - 1)
            for c in (0, 1):
                ww = fw if c else (1.0 - fw)
                wi = w0 + c
                valid = vd & vh & (wi >= 0) & (wi < W)
                cw = jnp.clip(wi, 0, W - 1)
                wt = wd * wh * ww * mk * valid.astype(jnp.float32)
                ci = (cd * (H * W) + ch * W + cw) * G + g_l
                iw_ref[:, corner, :] = ci
                iw_ref[:, NCORNER + corner, :] = lax.bitcast_convert_type(
                    wt, jnp.int32
                )
                corner += 1


def _prep(x, vwT, vb, omT_pad, omb_pad):
    bl = 512
    grid = L // bl
    return pl.pallas_call(
        _prep_body,
        grid=(grid,),
        in_specs=[
            pl.BlockSpec((bl, C), lambda i: (i, 0)),
            pl.BlockSpec((C, C), lambda i: (0, 0)),
            pl.BlockSpec((C,), lambda i: (0,)),
            pl.BlockSpec((C, 4 * LANES), lambda i: (0, 0)),
            pl.BlockSpec((4 * LANES,), lambda i: (0,)),
        ],
        out_specs=[
            pl.BlockSpec((bl, C), lambda i: (i, 0)),
            pl.BlockSpec((bl, 2 * NCORNER, LANES), lambda i: (i, 0, 0)),
        ],
        out_shape=[
            jax.ShapeDtypeStruct((L, C), jnp.float32),
            jax.ShapeDtypeStruct((L, 2 * NCORNER, LANES), jnp.int32),
        ],
    )(x, vwT, vb, omT_pad, omb_pad)


NBUF = 3


def _sc_body(value_hbm, iw_hbm, out_hbm, iw_v, rows_v, out_v,
             iwsem, gsem, osem):
    wid = lax.axis_index("s") * NC + lax.axis_index("c")
    base = wid * PER_W
    last = L - 1

    def iw_copy(l, b):
        return pltpu.async_copy(iw_hbm.at[l], iw_v.at[b], iwsem.at[b])

    def gather(l_unused, b):
        return [
            pltpu.async_copy(
                value_hbm.at[iw_v.at[b, cc]], rows_v.at[b, cc], gsem.at[b]
            )
            for cc in range(NCORNER)
        ]

    iw_copy(base, 0).wait()
    d_iw1 = iw_copy(jnp.minimum(base + 1, last), 1)
    gather(None, 0)
    d_iw1.wait()

    def step(t, _):
        for u in range(NBUF):
            i = NBUF * t + u
            l = base + i
            un = (u + 1) % NBUF
            up = (u + 2) % NBUF
            for cc in range(NCORNER):
                pltpu.make_async_copy(
                    value_hbm.at[iw_v.at[u, cc]], rows_v.at[u, cc], gsem.at[u]
                ).wait()
            gather(None, un)
            iw_copy(jnp.minimum(l + 2, last), up)
            @pl.when(i >= NBUF)
            def _():
                pltpu.make_async_copy(
                    out_v.at[u], out_hbm.at[l - NBUF], osem.at[u]
                ).wait()

            def per_corner(cc, accs):
                new = list(accs)
                for chunk in range((G * PTS + 15) // 16):
                    w16 = plsc.bitcast(
                        iw_v[u, NCORNER + cc, pl.ds(chunk * 16, 16)], jnp.float32
                    )
                    for j in range(16):
                        s = chunk * 16 + j
                        if s >= G * PTS:
                            break
                        g = s // PTS
                        wt = w16[j]
                        new[2 * g] = new[2 * g] + wt * rows_v[u, cc, s, pl.ds(0, 16)]
                        new[2 * g + 1] = (
                            new[2 * g + 1] + wt * rows_v[u, cc, s, pl.ds(16, 16)]
                        )
                return tuple(new)

            zero = jnp.zeros((16,), jnp.float32)
            accs = lax.fori_loop(0, NCORNER, per_corner, (zero,) * (2 * G))
            for g in range(G):
                out_v[u, pl.ds(g * GC, 16)] = accs[2 * g]
                out_v[u, pl.ds(g * GC + 16, 16)] = accs[2 * g + 1]
            pltpu.async_copy(out_v.at[u], out_hbm.at[l], osem.at[u])
            pltpu.make_async_copy(
                iw_hbm.at[0], iw_v.at[up], iwsem.at[up]
            ).wait()
        return 0

    lax.fori_loop(0, PER_W // NBUF, step, 0)
    for cc in range(NCORNER):
        pltpu.make_async_copy(
            value_hbm.at[iw_v.at[0, cc]], rows_v.at[0, cc], gsem.at[0]
        ).wait()
    for u in range(NBUF):
        l_tail = base + PER_W - NBUF + u
        pltpu.make_async_copy(out_v.at[u], out_hbm.at[l_tail], osem.at[u]).wait()


def _sc_sample(value_flat, iw):
    mesh = plsc.VectorSubcoreMesh(
        core_axis_name="c", subcore_axis_name="s", num_cores=NC, num_subcores=NS
    )
    return pl.kernel(
        _sc_body,
        out_type=jax.ShapeDtypeStruct((L, C), jnp.float32),
        mesh=mesh,
        compiler_params=pltpu.CompilerParams(
            use_tc_tiling_on_sc=False, needs_layout_passes=False
        ),
        scratch_types=[
            pltpu.VMEM((NBUF, 2 * NCORNER, LANES), jnp.int32),
            pltpu.VMEM((NBUF, NCORNER, LANES, GC), jnp.float32),
            pltpu.VMEM((NBUF, C), jnp.float32),
            pltpu.SemaphoreType.DMA((NBUF,)),
            pltpu.SemaphoreType.DMA((NBUF,)),
            pltpu.SemaphoreType.DMA((NBUF,)),
        ],
    )(value_flat, iw)


def _oproj_body(x_ref, owT_ref, ob_ref, out_ref):
    out_ref[...] = (
        jnp.dot(x_ref[...], owT_ref[...], preferred_element_type=jnp.float32)
        + ob_ref[...]
    )


def _oproj(x, owT, ob):
    bl = 512
    return pl.pallas_call(
        _oproj_body,
        grid=(L // bl,),
        in_specs=[
            pl.BlockSpec((bl, C), lambda i: (i, 0)),
            pl.BlockSpec((C, C), lambda i: (0, 0)),
            pl.BlockSpec((C,), lambda i: (0,)),
        ],
        out_specs=pl.BlockSpec((bl, C), lambda i: (i, 0)),
        out_shape=jax.ShapeDtypeStruct((L, C), jnp.float32),
    )(x, owT, ob)


def kernel(input, value_proj_w, value_proj_b, offset_mask_w, offset_mask_b,
           output_proj_w, output_proj_b):
    n, d, h, w_, c = input.shape
    x = input.reshape(L, C)
    perm = jnp.asarray(_PERM)
    live = jnp.asarray(_LIVE, dtype=jnp.float32)
    omw_pad = offset_mask_w[perm] * live[:, None]
    omb_pad = offset_mask_b[perm] * live

    value, iw = _prep(
        x, value_proj_w.T, value_proj_b, omw_pad.T, omb_pad
    )
    value_flat = value.reshape(L * G, GC)
    sampled = _sc_sample(value_flat, iw)
    out = _oproj(sampled, output_proj_w.T, output_proj_b)
    return out.reshape(n, d, h, w_, c)
